# baseline (speedup 1.0000x reference)
import jax
import jax.numpy as jnp
from jax import lax
from jax.experimental import pallas as pl
from jax.experimental.pallas import tpu as pltpu
from jax.experimental.pallas import tpu_sc as plsc

N = 10000
E = 320000
D = 128
DH = D // 2

NS = 16
EW = E // NS
C = 80
NCHUNK = EW // C


def _scores(h, a2_ref):
    return jnp.dot(h, a2_ref[...], preferred_element_type=jnp.float32)


def _pre_body(x_ref, w_ref, a2_ref, hl_ref, hr_ref, sc_ref):
    h = jnp.dot(x_ref[...], w_ref[...], preferred_element_type=jnp.float32)
    hl_ref[...] = h[:, :DH]
    hr_ref[...] = h[:, DH:]
    sc_ref[...] = _scores(h, a2_ref)


def _pre(x, W, a_src, a_dst):
    a2 = jnp.stack([a_src, a_dst], axis=1)
    bn = 2000
    return pl.pallas_call(
        _pre_body,
        grid=(N // bn,),
        in_specs=[
            pl.BlockSpec((bn, D), lambda i: (i, 0)),
            pl.BlockSpec((D, D), lambda i: (0, 0)),
            pl.BlockSpec((D, 2), lambda i: (0, 0)),
        ],
        out_specs=[
            pl.BlockSpec((bn, DH), lambda i: (i, 0)),
            pl.BlockSpec((bn, DH), lambda i: (i, 0)),
            pl.BlockSpec((bn, 2), lambda i: (i, 0)),
        ],
        out_shape=[
            jax.ShapeDtypeStruct((N, DH), jnp.float32),
            jax.ShapeDtypeStruct((N, DH), jnp.float32),
            jax.ShapeDtypeStruct((N, 2), jnp.float32),
        ],
    )(x, W, a2)


NBUF = 5
NGRP = NCHUNK // NBUF


def _edge_body(hl_hbm, hr_hbm, sc_hbm, src_hbm, dst_hbm, u_out, d0_out,
               d1_out, src_ts, dst_ts, sc_ts,
               w0, w1, w2, w3, w4, r0_, r1_, r2_, r3_, r4_,
               u_acc, den_acc,
               g0, g1, g2, g3, g4, s0, s1, s2, s3, s4, d0, d1, d2, d3, d4):
    cid = lax.axis_index("c")
    sid = lax.axis_index("s")
    w_list = (w0, w1, w2, w3, w4)
    rows_list = (r0_, r1_, r2_, r3_, r4_)
    gsems = (g0, g1, g2, g3, g4)
    ssems = (s0, s1, s2, s3, s4)
    dsems = (d0, d1, d2, d3, d4)

    z16 = jnp.zeros((16,), jnp.float32)

    def _zrow_fill(r, _):
        for q in range(DH // 16):
            r0_[r, pl.ds(q * 16, 16)] = z16
        return 0
    lax.fori_loop(0, C, _zrow_fill, 0)

    def _zden_fill(k, _):
        w0[pl.ds(k * 16, 16)] = z16
        return 0
    lax.fori_loop(0, C // 16, _zden_fill, 0)

    @pl.when(sid < 10)
    def _():
        for k in range(12):
            pltpu.sync_copy(r0_, u_acc.at[pl.ds(sid * 1000 + k * 80, 80)])
        pltpu.sync_copy(r0_.at[pl.ds(0, 40)],
                        u_acc.at[pl.ds(sid * 1000 + 960, 40)])

    @pl.when(sid < 5)
    def _():
        for k in range(25):
            pltpu.sync_copy(w0, den_acc.at[pl.ds(sid * 2000 + k * 80, 80)])

    pltpu.sync_copy(src_hbm.at[sid], src_ts)
    pltpu.sync_copy(dst_hbm.at[sid], dst_ts)
    pltpu.sync_copy(sc_hbm, sc_ts)
    plsc.subcore_barrier()

    one16 = jnp.ones((16,), jnp.int32)

    def _main_loop(h_ref, den_pred):
        def _wait_scat(b, ci):
            pltpu.make_async_copy(rows_list[b], u_acc.at[dst_ts.at[ci]],
                                  ssems[b]).wait()

            @pl.when(den_pred(ci))
            def _():
                pltpu.make_async_copy(w_list[b], den_acc.at[dst_ts.at[ci]],
                                      dsems[b]).wait()

        def _gather(ci, b):
            pltpu.async_copy(h_ref.at[src_ts.at[ci]], rows_list[b], gsems[b])

        _gather(0, 0)
        _gather(1, 1)

        def _group(g, _):
            for b in range(NBUF):
                ci = g * NBUF + b
                rows = rows_list[b]
                w_buf = w_list[b]
                pltpu.make_async_copy(h_ref.at[src_ts.at[ci]], rows,
                                      gsems[b]).wait()
                for j in range(C // 16):
                    sv = src_ts[ci, pl.ds(j * 16, 16)]
                    dv = dst_ts[ci, pl.ds(j * 16, 16)]
                    a = plsc.load_gather(sc_ts, [sv * 2])
                    bb = plsc.load_gather(sc_ts, [dv * 2 + one16])
                    e = a + bb
                    e = jnp.where(e >= 0.0, e, 0.2 * e)
                    w_buf[pl.ds(j * 16, 16)] = jnp.exp(e)

                @plsc.parallel_loop(0, C, step=1, unroll=8)
                def _row(r):
                    wspl = plsc.load_gather(
                        w_buf, [jnp.zeros((16,), jnp.int32) + r])
                    for q in range(DH // 16):
                        rows[r, pl.ds(q * 16, 16)] = (
                            rows[r, pl.ds(q * 16, 16)] * wspl)

                pltpu.async_copy(rows, u_acc.at[dst_ts.at[ci]], ssems[b],
                                 add=True)

                @pl.when(den_pred(ci))
                def _():
                    pltpu.async_copy(w_buf, den_acc.at[dst_ts.at[ci]],
                                     dsems[b], add=True)

                bn = (b + 2) % NBUF

                def _prefetch():
                    _gather(ci + 2, bn)

                if b < 3:
                    @pl.when(g > 0)
                    def _():
                        _wait_scat(bn, ci - 3)
                    _prefetch()
                else:
                    _wait_scat(bn, ci - 3)

                    @pl.when(g < NGRP - 1)
                    def _():
                        _prefetch()
            return 0
        lax.fori_loop(0, NGRP, _group, 0)

        for b in (2, 3, 4):
            _wait_scat(b, NCHUNK - NBUF + b)

    half = NCHUNK // 2

    @pl.when(cid == 0)
    def _():
        _main_loop(hl_hbm, lambda ci: ci < half)

    @pl.when(cid == 1)
    def _():
        _main_loop(hr_hbm, lambda ci: ci >= half)

    plsc.subcore_barrier()

    @pl.when(sid < 10)
    def _():
        for k in range(12):
            rb = r0_ if k % 2 == 0 else r1_
            ro = sid * 1000 + k * 80
            pltpu.sync_copy(u_acc.at[pl.ds(ro, 80)], rb)
            pltpu.sync_copy(rb, u_out.at[cid, pl.ds(ro, 80)])
        ro = sid * 1000 + 960
        pltpu.sync_copy(u_acc.at[pl.ds(ro, 40)], r2_.at[pl.ds(0, 40)])
        pltpu.sync_copy(r2_.at[pl.ds(0, 40)], u_out.at[cid, pl.ds(ro, 40)])

    @pl.when(sid < 5)
    def _():
        for k in range(25):
            wb = w0 if k % 2 == 0 else w1
            ro = sid * 2000 + k * 80
            pltpu.sync_copy(den_acc.at[pl.ds(ro, 80)], wb)

            @pl.when(cid == 0)
            def _():
                pltpu.sync_copy(wb, d0_out.at[pl.ds(ro, 80)])

            @pl.when(cid == 1)
            def _():
                pltpu.sync_copy(wb, d1_out.at[pl.ds(ro, 80)])


_edge_call = pl.kernel(
    _edge_body,
    out_type=[
        jax.ShapeDtypeStruct((2, N, DH), jnp.float32),
        jax.ShapeDtypeStruct((N,), jnp.float32),
        jax.ShapeDtypeStruct((N,), jnp.float32),
    ],
    mesh=plsc.VectorSubcoreMesh(core_axis_name="c", subcore_axis_name="s"),
    scratch_types=[
        pltpu.VMEM((NCHUNK, C), jnp.int32),
        pltpu.VMEM((NCHUNK, C), jnp.int32),
        pltpu.VMEM((2 * N,), jnp.float32),
    ] + [pltpu.VMEM((C,), jnp.float32) for _ in range(NBUF)]
      + [pltpu.VMEM((C, DH), jnp.float32) for _ in range(NBUF)]
      + [
        pltpu.VMEM_SHARED((N, DH), jnp.float32),
        pltpu.VMEM_SHARED((N,), jnp.float32),
    ] + [pltpu.SemaphoreType.DMA for _ in range(3 * NBUF)],
    compiler_params=pltpu.CompilerParams(needs_layout_passes=False,
                                         use_tc_tiling_on_sc=False),
)


def _norm_out(u2_ref, den_ref, hl_ref, hr_ref, sc_ref, b_ref, g_ref, be_ref,
              a_ref):
    h = jnp.concatenate([hl_ref[...], hr_ref[...]], axis=1)
    sc = sc_ref[...]
    wself = jnp.exp(jax.nn.leaky_relu(sc[:, 0] + sc[:, 1], 0.2))
    u = jnp.concatenate([u2_ref[0], u2_ref[1]], axis=1) + wself[:, None] * h
    den = den_ref[0, :] + den_ref[1, :] + wself
    o = u / (den[:, None] + 1e-16) + b_ref[...]
    m = jnp.mean(o)
    s = jnp.sqrt(jnp.mean((o - m) ** 2))
    o = (o - m) / (s + 1e-5) * g_ref[...] + be_ref[...]
    a = a_ref[0, 0]
    return jnp.where(o >= 0.0, o, a * o)


def _combpre_body(u2_ref, den_ref, hl_ref, hr_ref, sc_ref, b_ref, g_ref,
                  be_ref, a_ref, w_ref, a2_ref, hl2_ref, hr2_ref, sc2_ref):
    o = _norm_out(u2_ref, den_ref, hl_ref, hr_ref, sc_ref, b_ref, g_ref,
                  be_ref, a_ref)
    h2 = jnp.dot(o, w_ref[...], preferred_element_type=jnp.float32)
    hl2_ref[...] = h2[:, :DH]
    hr2_ref[...] = h2[:, DH:]
    sc2_ref[...] = _scores(h2, a2_ref)


def _combpre(u2, den, hl, hr, sc, b, g, be, a, W2, as2, ad2):
    a2 = jnp.stack([as2, ad2], axis=1)
    return pl.pallas_call(
        _combpre_body,
        out_shape=[
            jax.ShapeDtypeStruct((N, DH), jnp.float32),
            jax.ShapeDtypeStruct((N, DH), jnp.float32),
            jax.ShapeDtypeStruct((N, 2), jnp.float32),
        ],
    )(u2, den, hl, hr, sc,
      b.reshape(1, D), g.reshape(1, D), be.reshape(1, D), a.reshape(1, 1),
      W2, a2)


def _combfin_body(u2_ref, den_ref, hl_ref, hr_ref, sc_ref, b_ref, g_ref,
                  be_ref, a_ref, o_ref):
    o_ref[...] = _norm_out(u2_ref, den_ref, hl_ref, hr_ref, sc_ref, b_ref,
                           g_ref, be_ref, a_ref)


def _combfin(u2, den, hl, hr, sc, b, g, be, a):
    return pl.pallas_call(
        _combfin_body,
        out_shape=jax.ShapeDtypeStruct((N, D), jnp.float32),
    )(u2, den, hl, hr, sc,
      b.reshape(1, D), g.reshape(1, D), be.reshape(1, D), a.reshape(1, 1))


def _combmap_body(u2_ref, den_ref, hl_ref, hr_ref, sc_ref, b_ref, g_ref,
                  be_ref, a_ref, m1_ref, mb1_ref, mp_ref, m2_ref, mb2_ref,
                  o_ref):
    o = _norm_out(u2_ref, den_ref, hl_ref, hr_ref, sc_ref, b_ref, g_ref,
                  be_ref, a_ref)
    hm = jnp.dot(o, m1_ref[...], preferred_element_type=jnp.float32)
    hm = hm + mb1_ref[...]
    am = mp_ref[0, 0]
    hm = jnp.where(hm >= 0.0, hm, am * hm)
    om = jnp.dot(hm, m2_ref[...], preferred_element_type=jnp.float32)
    o_ref[...] = om + mb2_ref[...]


def _combmap(u2, den, hl, hr, sc, b, g, be, a, pm):
    return pl.pallas_call(
        _combmap_body,
        out_shape=jax.ShapeDtypeStruct((N, D), jnp.float32),
    )(u2, den, hl, hr, sc,
      b.reshape(1, D), g.reshape(1, D), be.reshape(1, D), a.reshape(1, 1),
      pm['M1'], pm['mb1'].reshape(1, -1), pm['mp'].reshape(1, 1),
      pm['M2'], pm['mb2'].reshape(1, -1))


def _encoder(x, edge_index, p, pm=None):
    src3 = edge_index[0].reshape(NS, NCHUNK, C)
    dst3 = edge_index[1].reshape(NS, NCHUNK, C)
    hl, hr, sc = _pre(x, p['W1'], p['as1'], p['ad1'])
    u2, da, db = _edge_call(hl, hr, sc.reshape(2 * N), src3, dst3)
    den = jnp.stack([da, db])
    hl2, hr2, sc2 = _combpre(u2, den, hl, hr, sc, p['b1'], p['g1'], p['be1'],
                             p['p1'], p['W2'], p['as2'], p['ad2'])
    u2b, da2, db2 = _edge_call(hl2, hr2, sc2.reshape(2 * N), src3, dst3)
    den2 = jnp.stack([da2, db2])
    if pm is None:
        return _combfin(u2b, den2, hl2, hr2, sc2, p['b2'], p['g2'], p['be2'],
                        p['p2'])
    return _combmap(u2b, den2, hl2, hr2, sc2, p['b2'], p['g2'], p['be2'],
                    p['p2'], pm)


def kernel(x_q, edge_index_q, x_k, edge_index_k, params_q, params_k, params_m):
    Q = _encoder(x_q, edge_index_q, params_q, params_m)
    K = _encoder(x_k, edge_index_k, params_k)
    return (Q, K)


# speedup vs baseline: 1.2435x; 1.2435x over previous
import jax
import jax.numpy as jnp
from jax import lax
from jax.experimental import pallas as pl
from jax.experimental.pallas import tpu as pltpu
from jax.experimental.pallas import tpu_sc as plsc

N = 10000
E = 320000
D = 128
DH = D // 2

NS = 16
EW = E // NS
C = 80
NCHUNK = EW // C


def _scores(h, a2_ref):
    return jnp.dot(h, a2_ref[...], preferred_element_type=jnp.float32)


def _pre_body(x_ref, w_ref, a2_ref, hl_ref, hr_ref, sc_ref):
    h = jnp.dot(x_ref[...], w_ref[...], preferred_element_type=jnp.float32)
    hl_ref[...] = h[:, :DH]
    hr_ref[...] = h[:, DH:]
    sc_ref[...] = _scores(h, a2_ref)


def _pre(x, W, a_src, a_dst):
    a2 = jnp.stack([a_src, a_dst], axis=1)
    bn = 2000
    return pl.pallas_call(
        _pre_body,
        grid=(N // bn,),
        in_specs=[
            pl.BlockSpec((bn, D), lambda i: (i, 0)),
            pl.BlockSpec((D, D), lambda i: (0, 0)),
            pl.BlockSpec((D, 2), lambda i: (0, 0)),
        ],
        out_specs=[
            pl.BlockSpec((bn, DH), lambda i: (i, 0)),
            pl.BlockSpec((bn, DH), lambda i: (i, 0)),
            pl.BlockSpec((bn, 2), lambda i: (i, 0)),
        ],
        out_shape=[
            jax.ShapeDtypeStruct((N, DH), jnp.float32),
            jax.ShapeDtypeStruct((N, DH), jnp.float32),
            jax.ShapeDtypeStruct((N, 2), jnp.float32),
        ],
    )(x, W, a2)


NBUF = 5
NGRP = NCHUNK // NBUF


def _edge_body(hl_hbm, hr_hbm, sc_hbm, src_hbm, dst_hbm, u_out, d0_out,
               d1_out, src_ts, dst_ts, sc_ts,
               w0, w1, w2, w3, w4, r0_, r1_, r2_, r3_, r4_,
               u_acc, den_acc,
               g0, g1, g2, g3, g4, s0, s1, s2, s3, s4, d0, d1, d2, d3, d4):
    cid = lax.axis_index("c")
    sid = lax.axis_index("s")
    w_list = (w0, w1, w2, w3, w4)
    rows_list = (r0_, r1_, r2_, r3_, r4_)
    gsems = (g0, g1, g2, g3, g4)
    ssems = (s0, s1, s2, s3, s4)
    dsems = (d0, d1, d2, d3, d4)

    z16 = jnp.zeros((16,), jnp.float32)

    def _zrow_fill(r, _):
        for q in range(DH // 16):
            r0_[r, pl.ds(q * 16, 16)] = z16
        return 0
    lax.fori_loop(0, C, _zrow_fill, 0)

    def _zden_fill(k, _):
        w0[pl.ds(k * 16, 16)] = z16
        return 0
    lax.fori_loop(0, C // 16, _zden_fill, 0)

    @pl.when(sid < 10)
    def _():
        for k in range(12):
            pltpu.sync_copy(r0_, u_acc.at[pl.ds(sid * 1000 + k * 80, 80)])
        pltpu.sync_copy(r0_.at[pl.ds(0, 40)],
                        u_acc.at[pl.ds(sid * 1000 + 960, 40)])

    @pl.when(sid < 5)
    def _():
        for k in range(25):
            pltpu.sync_copy(w0, den_acc.at[pl.ds(sid * 2000 + k * 80, 80)])

    pltpu.sync_copy(src_hbm.at[sid], src_ts)
    pltpu.sync_copy(dst_hbm.at[sid], dst_ts)
    pltpu.sync_copy(sc_hbm, sc_ts)
    plsc.subcore_barrier()

    one16 = jnp.ones((16,), jnp.int32)

    def _main_loop(h_ref, den_pred):
        def _wait_scat(b, ci):
            pltpu.make_async_copy(rows_list[b], u_acc.at[dst_ts.at[ci]],
                                  ssems[b]).wait()

            @pl.when(den_pred(ci))
            def _():
                pltpu.make_async_copy(w_list[b], den_acc.at[dst_ts.at[ci]],
                                      dsems[b]).wait()

        def _gather(ci, b):
            pltpu.async_copy(h_ref.at[src_ts.at[ci]], rows_list[b], gsems[b])

        _gather(0, 0)
        _gather(1, 1)
        _gather(2, 2)

        def _group(g, _):
            for b in range(NBUF):
                ci = g * NBUF + b
                rows = rows_list[b]
                w_buf = w_list[b]
                pltpu.make_async_copy(h_ref.at[src_ts.at[ci]], rows,
                                      gsems[b]).wait()
                for j in range(C // 16):
                    sv = src_ts[ci, pl.ds(j * 16, 16)]
                    dv = dst_ts[ci, pl.ds(j * 16, 16)]
                    a = plsc.load_gather(sc_ts, [sv * 2])
                    bb = plsc.load_gather(sc_ts, [dv * 2 + one16])
                    e = a + bb
                    e = jnp.where(e >= 0.0, e, 0.2 * e)
                    w_buf[pl.ds(j * 16, 16)] = jnp.exp(e)

                @plsc.parallel_loop(0, C, step=1, unroll=8)
                def _row(r):
                    wspl = plsc.load_gather(
                        w_buf, [jnp.zeros((16,), jnp.int32) + r])
                    for q in range(DH // 16):
                        rows[r, pl.ds(q * 16, 16)] = (
                            rows[r, pl.ds(q * 16, 16)] * wspl)

                pltpu.async_copy(rows, u_acc.at[dst_ts.at[ci]], ssems[b],
                                 add=True)

                @pl.when(den_pred(ci))
                def _():
                    pltpu.async_copy(w_buf, den_acc.at[dst_ts.at[ci]],
                                     dsems[b], add=True)

                bn = (b + 3) % NBUF

                def _prefetch():
                    _gather(ci + 3, bn)

                if b < 2:
                    @pl.when(g > 0)
                    def _():
                        _wait_scat(bn, ci - 2)
                    _prefetch()
                else:
                    _wait_scat(bn, ci - 2)

                    @pl.when(g < NGRP - 1)
                    def _():
                        _prefetch()
            return 0
        lax.fori_loop(0, NGRP, _group, 0)

        for b in (3, 4):
            _wait_scat(b, NCHUNK - NBUF + b)

    half = NCHUNK // 2

    @pl.when(cid == 0)
    def _():
        _main_loop(hl_hbm, lambda ci: ci < half)

    @pl.when(cid == 1)
    def _():
        _main_loop(hr_hbm, lambda ci: ci >= half)

    plsc.subcore_barrier()

    @pl.when(sid < 10)
    def _():
        for k in range(12):
            rb = r0_ if k % 2 == 0 else r1_
            ro = sid * 1000 + k * 80
            pltpu.sync_copy(u_acc.at[pl.ds(ro, 80)], rb)
            pltpu.sync_copy(rb, u_out.at[cid, pl.ds(ro, 80)])
        ro = sid * 1000 + 960
        pltpu.sync_copy(u_acc.at[pl.ds(ro, 40)], r2_.at[pl.ds(0, 40)])
        pltpu.sync_copy(r2_.at[pl.ds(0, 40)], u_out.at[cid, pl.ds(ro, 40)])

    @pl.when(sid < 5)
    def _():
        for k in range(25):
            wb = w0 if k % 2 == 0 else w1
            ro = sid * 2000 + k * 80
            pltpu.sync_copy(den_acc.at[pl.ds(ro, 80)], wb)

            @pl.when(cid == 0)
            def _():
                pltpu.sync_copy(wb, d0_out.at[pl.ds(ro, 80)])

            @pl.when(cid == 1)
            def _():
                pltpu.sync_copy(wb, d1_out.at[pl.ds(ro, 80)])


_edge_call = pl.kernel(
    _edge_body,
    out_type=[
        jax.ShapeDtypeStruct((2, N, DH), jnp.float32),
        jax.ShapeDtypeStruct((N,), jnp.float32),
        jax.ShapeDtypeStruct((N,), jnp.float32),
    ],
    mesh=plsc.VectorSubcoreMesh(core_axis_name="c", subcore_axis_name="s"),
    scratch_types=[
        pltpu.VMEM((NCHUNK, C), jnp.int32),
        pltpu.VMEM((NCHUNK, C), jnp.int32),
        pltpu.VMEM((2 * N,), jnp.float32),
    ] + [pltpu.VMEM((C,), jnp.float32) for _ in range(NBUF)]
      + [pltpu.VMEM((C, DH), jnp.float32) for _ in range(NBUF)]
      + [
        pltpu.VMEM_SHARED((N, DH), jnp.float32),
        pltpu.VMEM_SHARED((N,), jnp.float32),
    ] + [pltpu.SemaphoreType.DMA for _ in range(3 * NBUF)],
    compiler_params=pltpu.CompilerParams(needs_layout_passes=False,
                                         use_tc_tiling_on_sc=False),
)


def _norm_out(u2_ref, den_ref, hl_ref, hr_ref, sc_ref, b_ref, g_ref, be_ref,
              a_ref):
    h = jnp.concatenate([hl_ref[...], hr_ref[...]], axis=1)
    sc = sc_ref[...]
    wself = jnp.exp(jax.nn.leaky_relu(sc[:, 0] + sc[:, 1], 0.2))
    u = jnp.concatenate([u2_ref[0], u2_ref[1]], axis=1) + wself[:, None] * h
    den = den_ref[0, :] + den_ref[1, :] + wself
    o = u / (den[:, None] + 1e-16) + b_ref[...]
    m = jnp.mean(o)
    s = jnp.sqrt(jnp.mean((o - m) ** 2))
    o = (o - m) / (s + 1e-5) * g_ref[...] + be_ref[...]
    a = a_ref[0, 0]
    return jnp.where(o >= 0.0, o, a * o)


def _combpre_body(u2_ref, den_ref, hl_ref, hr_ref, sc_ref, b_ref, g_ref,
                  be_ref, a_ref, w_ref, a2_ref, hl2_ref, hr2_ref, sc2_ref):
    o = _norm_out(u2_ref, den_ref, hl_ref, hr_ref, sc_ref, b_ref, g_ref,
                  be_ref, a_ref)
    h2 = jnp.dot(o, w_ref[...], preferred_element_type=jnp.float32)
    hl2_ref[...] = h2[:, :DH]
    hr2_ref[...] = h2[:, DH:]
    sc2_ref[...] = _scores(h2, a2_ref)


def _combpre(u2, den, hl, hr, sc, b, g, be, a, W2, as2, ad2):
    a2 = jnp.stack([as2, ad2], axis=1)
    return pl.pallas_call(
        _combpre_body,
        out_shape=[
            jax.ShapeDtypeStruct((N, DH), jnp.float32),
            jax.ShapeDtypeStruct((N, DH), jnp.float32),
            jax.ShapeDtypeStruct((N, 2), jnp.float32),
        ],
    )(u2, den, hl, hr, sc,
      b.reshape(1, D), g.reshape(1, D), be.reshape(1, D), a.reshape(1, 1),
      W2, a2)


def _combfin_body(u2_ref, den_ref, hl_ref, hr_ref, sc_ref, b_ref, g_ref,
                  be_ref, a_ref, o_ref):
    o_ref[...] = _norm_out(u2_ref, den_ref, hl_ref, hr_ref, sc_ref, b_ref,
                           g_ref, be_ref, a_ref)


def _combfin(u2, den, hl, hr, sc, b, g, be, a):
    return pl.pallas_call(
        _combfin_body,
        out_shape=jax.ShapeDtypeStruct((N, D), jnp.float32),
    )(u2, den, hl, hr, sc,
      b.reshape(1, D), g.reshape(1, D), be.reshape(1, D), a.reshape(1, 1))


def _combmap_body(u2_ref, den_ref, hl_ref, hr_ref, sc_ref, b_ref, g_ref,
                  be_ref, a_ref, m1_ref, mb1_ref, mp_ref, m2_ref, mb2_ref,
                  o_ref):
    o = _norm_out(u2_ref, den_ref, hl_ref, hr_ref, sc_ref, b_ref, g_ref,
                  be_ref, a_ref)
    hm = jnp.dot(o, m1_ref[...], preferred_element_type=jnp.float32)
    hm = hm + mb1_ref[...]
    am = mp_ref[0, 0]
    hm = jnp.where(hm >= 0.0, hm, am * hm)
    om = jnp.dot(hm, m2_ref[...], preferred_element_type=jnp.float32)
    o_ref[...] = om + mb2_ref[...]


def _combmap(u2, den, hl, hr, sc, b, g, be, a, pm):
    return pl.pallas_call(
        _combmap_body,
        out_shape=jax.ShapeDtypeStruct((N, D), jnp.float32),
    )(u2, den, hl, hr, sc,
      b.reshape(1, D), g.reshape(1, D), be.reshape(1, D), a.reshape(1, 1),
      pm['M1'], pm['mb1'].reshape(1, -1), pm['mp'].reshape(1, 1),
      pm['M2'], pm['mb2'].reshape(1, -1))


def _encoder(x, edge_index, p, pm=None):
    src3 = edge_index[0].reshape(NS, NCHUNK, C)
    dst3 = edge_index[1].reshape(NS, NCHUNK, C)
    hl, hr, sc = _pre(x, p['W1'], p['as1'], p['ad1'])
    u2, da, db = _edge_call(hl, hr, sc.reshape(2 * N), src3, dst3)
    den = jnp.stack([da, db])
    hl2, hr2, sc2 = _combpre(u2, den, hl, hr, sc, p['b1'], p['g1'], p['be1'],
                             p['p1'], p['W2'], p['as2'], p['ad2'])
    u2b, da2, db2 = _edge_call(hl2, hr2, sc2.reshape(2 * N), src3, dst3)
    den2 = jnp.stack([da2, db2])
    if pm is None:
        return _combfin(u2b, den2, hl2, hr2, sc2, p['b2'], p['g2'], p['be2'],
                        p['p2'])
    return _combmap(u2b, den2, hl2, hr2, sc2, p['b2'], p['g2'], p['be2'],
                    p['p2'], pm)


def kernel(x_q, edge_index_q, x_k, edge_index_k, params_q, params_k, params_m):
    Q = _encoder(x_q, edge_index_q, params_q, params_m)
    K = _encoder(x_k, edge_index_k, params_k)
    return (Q, K)
